# Initial kernel scaffold; baseline (speedup 1.0000x reference)
import jax
import jax.numpy as jnp
from jax.experimental import pallas as pl


def kernel(h, x, edge_index, params):
    raise NotImplementedError("write your pallas kernel here")


# speedup vs baseline: 1.6466x; 1.6466x over previous
import functools

import jax
import jax.numpy as jnp
from jax import lax
from jax.experimental import pallas as pl
from jax.experimental.pallas import tpu as pltpu
from jax.experimental.pallas import tpu_sc as plsc

N = 10000
HID = 128
E = 320000

NC = 2
NS = 16
NW = NC * NS
CH = 128
NCHUNK = 79
EPT = NCHUNK * CH
E_PAD = NW * EPT
NACC = 10240
EB = 512
NB = 1000

@functools.lru_cache(maxsize=None)
def _mesh():
    return plsc.VectorSubcoreMesh(core_axis_name="c", subcore_axis_name="s",
                                  num_cores=NC, num_subcores=NS)


def _silu(v):
    return v * jax.nn.sigmoid(v)


def _make_gather(with_x):
    out_type = [jax.ShapeDtypeStruct((E_PAD, HID), jnp.float32)]
    scratch = [
        pltpu.VMEM((CH,), jnp.int32),
        pltpu.VMEM((CH,), jnp.int32),
        pltpu.VMEM((CH, HID), jnp.float32),
        pltpu.SemaphoreType.DMA,
    ]
    if with_x:
        out_type.append(jax.ShapeDtypeStruct((E_PAD * 8,), jnp.float32))
        scratch.append(pltpu.VMEM((CH * 8,), jnp.float32))
        scratch.append(pltpu.VMEM((N * 8,), jnp.float32))

    def body(rowg, colg, a_t, b_t, *rest):
        if with_x:
            x8f, z_out, d_out, idxr, idxc, bufz, sem, bufd, xv = rest
        else:
            z_out, idxr, idxc, bufz, sem = rest
        wid = lax.axis_index("s") * NC + lax.axis_index("c")
        base = wid * EPT
        if with_x:
            pltpu.sync_copy(x8f, xv)
            lanes8 = lax.iota(jnp.int32, 16) * 8

        def step(i, carry):
            off = base + i * CH
            pltpu.sync_copy(rowg.at[pl.ds(off, CH)], idxr)
            pltpu.sync_copy(colg.at[pl.ds(off, CH)], idxc)
            pltpu.async_copy(a_t.at[idxr], bufz, sem).wait()
            pltpu.async_copy(b_t.at[idxc], bufz, sem, add=True).wait()
            if with_x:
                for g in range(CH // 16):
                    ir = idxr[pl.ds(g * 16, 16)] * 8
                    ic = idxc[pl.ds(g * 16, 16)] * 8
                    for j in range(8):
                        vr = plsc.load_gather(xv, [ir + j])
                        vc = plsc.load_gather(xv, [ic + j])
                        plsc.store_scatter(bufd, [lanes8 + (g * 128 + j)],
                                           vr - vc)
                pltpu.sync_copy(bufd, d_out.at[pl.ds(off * 8, CH * 8)])
            pltpu.sync_copy(bufz, z_out.at[pl.ds(off, CH)])
            return carry

        lax.fori_loop(0, NCHUNK, step, 0)

    return pl.kernel(
        body, out_type=tuple(out_type), mesh=_mesh(), scratch_types=scratch,
        compiler_params=pltpu.CompilerParams(needs_layout_passes=False))


def _make_scatter(width):
    zpc = NACC // NS // CH

    def body(rows_hbm, m_hbm, out_hbm, idx, buf, acc, sem):
        c = lax.axis_index("c")
        s = lax.axis_index("s")
        wid = s * NC + c

        def zrow(i, carry):
            for j in range(width // 16):
                buf[i, pl.ds(j * 16, 16)] = jnp.zeros((16,), jnp.float32)
            return carry
        lax.fori_loop(0, CH, zrow, 0)

        def zacc(k, carry):
            pltpu.sync_copy(buf, acc.at[pl.ds((s * zpc + k) * CH, CH)])
            return carry
        lax.fori_loop(0, zpc, zacc, 0)
        plsc.subcore_barrier()

        base = wid * EPT

        def step(i, carry):
            off = base + i * CH
            pltpu.sync_copy(rows_hbm.at[pl.ds(off, CH)], idx)
            pltpu.sync_copy(m_hbm.at[pl.ds(off, CH)], buf)
            pltpu.sync_copy(buf, acc.at[idx], add=True)
            return carry
        lax.fori_loop(0, NCHUNK, step, 0)
        plsc.subcore_barrier()

        def cout(k, carry):
            r = (s * zpc + k) * CH
            pltpu.sync_copy(acc.at[pl.ds(r, CH)], out_hbm.at[c, pl.ds(r, CH)])
            return carry
        lax.fori_loop(0, zpc, cout, 0)

    return pl.kernel(
        body,
        out_type=jax.ShapeDtypeStruct((NC, NACC, width), jnp.float32),
        mesh=_mesh(),
        scratch_types=[
            pltpu.VMEM((CH,), jnp.int32),
            pltpu.VMEM((CH, width), jnp.float32),
            pltpu.VMEM_SHARED((NACC, width), jnp.float32),
            pltpu.SemaphoreType.DMA,
        ])


_make_gather = functools.lru_cache(maxsize=None)(_make_gather)
_make_scatter = functools.lru_cache(maxsize=None)(_make_scatter)


def _gather_x(*a):
    return _make_gather(True)(*a)


def _gather(*a):
    return _make_gather(False)(*a)[0]


def _scatter_h(*a):
    return _make_scatter(HID)(*a)


def _scatter_x(*a):
    return _make_scatter(HID)(*a)[:, :, :8]


def _wspec(shape):
    return pl.BlockSpec(shape, lambda i: (0,) * len(shape))


def _nspec(width=HID):
    return pl.BlockSpec((NB, width), lambda i: (i, 0))


def _espec(width=HID):
    return pl.BlockSpec((EB, width), lambda i: (i, 0))


def _tc_pre0(h, we1, be1, we2, be2, w1a, w1b):
    def body(h_r, we1_r, be1_r, we2_r, be2_r, w1a_r, w1b_r,
             h0_r, ha_r, hb_r):
        h0 = _silu(jnp.dot(h_r[...], we1_r[...],
                           preferred_element_type=jnp.float32, precision=lax.Precision.HIGHEST) + be1_r[...])
        h0 = jnp.dot(h0, we2_r[...], preferred_element_type=jnp.float32, precision=lax.Precision.HIGHEST) + be2_r[...]
        h0_r[...] = h0
        ha_r[...] = jnp.dot(h0, w1a_r[...], preferred_element_type=jnp.float32, precision=lax.Precision.HIGHEST)
        hb_r[...] = jnp.dot(h0, w1b_r[...], preferred_element_type=jnp.float32, precision=lax.Precision.HIGHEST)

    return pl.pallas_call(
        body,
        grid=(N // NB,),
        in_specs=[_nspec(), _wspec((HID, HID)), _wspec((1, HID)),
                  _wspec((HID, HID)), _wspec((1, HID)),
                  _wspec((HID, HID)), _wspec((HID, HID))],
        out_specs=[_nspec(), _nspec(), _nspec()],
        out_shape=[jax.ShapeDtypeStruct((N, HID), jnp.float32)] * 3,
    )(h, we1, be1, we2, be2, w1a, w1b)


def _tc_edge(first_block, z, d8, r0, wd, wd0, b1, w2, b2):
    def body(*refs):
        if first_block:
            (z_r, d8_r, wd_r, wd0_r, b1_r, w2_r, b2_r, m_r, r8_r) = refs
        else:
            (z_r, d8_r, r0_r, wd_r, wd0_r, b1_r, w2_r, b2_r, m_r) = refs
        d8 = d8_r[...]
        r = jnp.sum(d8 * d8, axis=1, keepdims=True)
        r0v = r if first_block else r0_r[:, 0:1]
        z1 = z_r[...] + r * wd_r[...] + r0v * wd0_r[...] + b1_r[...]
        a1 = _silu(z1)
        m_r[...] = _silu(jnp.dot(a1, w2_r[...],
                                 preferred_element_type=jnp.float32, precision=lax.Precision.HIGHEST) + b2_r[...])
        if first_block:
            r8_r[...] = jnp.broadcast_to(r, (EB, 8))

    in_specs = [_espec(), _espec(8)]
    ins = [z, d8]
    if not first_block:
        in_specs.append(_espec(8))
        ins.append(r0)
    in_specs += [_wspec((1, HID))] * 2 + [_wspec((1, HID)),
                                          _wspec((HID, HID)), _wspec((1, HID))]
    ins += [wd, wd0, b1, w2, b2]
    out_specs = [_espec()]
    out_shape = [jax.ShapeDtypeStruct((E_PAD, HID), jnp.float32)]
    if first_block:
        out_specs.append(_espec(8))
        out_shape.append(jax.ShapeDtypeStruct((E_PAD, 8), jnp.float32))

    res = pl.pallas_call(
        body, grid=(E_PAD // EB,), in_specs=in_specs,
        out_specs=out_specs, out_shape=out_shape)(*ins)
    return res if first_block else res[0]


def _tc_node(h, p0, p1, wh, wa, b1, w2, b2, c1a, c1b):
    def body(h_r, p0_r, p1_r, wh_r, wa_r, b1_r, w2_r, b2_r, c1a_r, c1b_r,
             hn_r, ca_r, cb_r):
        h0 = h_r[...]
        agg = (p0_r[...] + p1_r[...]) * 0.01
        t = _silu(jnp.dot(h0, wh_r[...], preferred_element_type=jnp.float32, precision=lax.Precision.HIGHEST)
                  + jnp.dot(agg, wa_r[...], preferred_element_type=jnp.float32, precision=lax.Precision.HIGHEST)
                  + b1_r[...])
        hn = h0 + jnp.dot(t, w2_r[...],
                          preferred_element_type=jnp.float32, precision=lax.Precision.HIGHEST) + b2_r[...]
        hn_r[...] = hn
        ca_r[...] = jnp.dot(hn, c1a_r[...], preferred_element_type=jnp.float32, precision=lax.Precision.HIGHEST)
        cb_r[...] = jnp.dot(hn, c1b_r[...], preferred_element_type=jnp.float32, precision=lax.Precision.HIGHEST)

    return pl.pallas_call(
        body, grid=(N // NB,),
        in_specs=[_nspec()] * 3 + [_wspec((HID, HID)), _wspec((HID, HID)),
                                   _wspec((1, HID)), _wspec((HID, HID)),
                                   _wspec((1, HID)), _wspec((HID, HID)),
                                   _wspec((HID, HID))],
        out_specs=[_nspec()] * 3,
        out_shape=[jax.ShapeDtypeStruct((N, HID), jnp.float32)] * 3,
    )(h, p0, p1, wh, wa, b1, w2, b2, c1a, c1b)


def _tc_coord(first_block, zc, d8, r0, wd, wd0, b1, w2, b2, wl):
    def body(*refs):
        if first_block:
            (zc_r, d8_r, wd_r, wd0_r, b1_r, w2_r, b2_r, wl_r, t8_r) = refs
        else:
            (zc_r, d8_r, r0_r, wd_r, wd0_r, b1_r, w2_r, b2_r, wl_r, t8_r) = refs
        d8 = d8_r[...]
        r = jnp.sum(d8 * d8, axis=1, keepdims=True)
        r0v = r if first_block else r0_r[:, 0:1]
        z1 = zc_r[...] + r * wd_r[...] + r0v * wd0_r[...] + b1_r[...]
        a2 = _silu(jnp.dot(_silu(z1), w2_r[...],
                           preferred_element_type=jnp.float32, precision=lax.Precision.HIGHEST) + b2_r[...])
        phi = jnp.sum(a2 * wl_r[...], axis=1, keepdims=True)
        cd8 = d8 / (jnp.sqrt(r + 1e-8) + 1.0)
        t8_r[...] = jnp.concatenate(
            [cd8 * phi, jnp.zeros((EB, HID - 8), jnp.float32)], axis=1)

    in_specs = [_espec(), _espec(8)]
    ins = [zc, d8]
    if not first_block:
        in_specs.append(_espec(8))
        ins.append(r0)
    in_specs += [_wspec((1, HID))] * 3 + [_wspec((HID, HID)),
                                          _wspec((1, HID)), _wspec((1, HID))]
    ins += [wd, wd0, b1, w2, b2, wl]

    return pl.pallas_call(
        body, grid=(E_PAD // EB,), in_specs=in_specs,
        out_specs=_espec(),
        out_shape=jax.ShapeDtypeStruct((E_PAD, HID), jnp.float32))(*ins)


def _tc_pre1(x8, xp0, xp1, h1, w1a, w1b):
    def body(x8_r, xp0_r, xp1_r, h1_r, w1a_r, w1b_r,
             x8n_r, ha_r, hb_r):
        xn = x8_r[...] + (xp0_r[...] + xp1_r[...]) * 0.01
        x8n_r[...] = xn
        h1 = h1_r[...]
        ha_r[...] = jnp.dot(h1, w1a_r[...], preferred_element_type=jnp.float32, precision=lax.Precision.HIGHEST)
        hb_r[...] = jnp.dot(h1, w1b_r[...], preferred_element_type=jnp.float32, precision=lax.Precision.HIGHEST)

    return pl.pallas_call(
        body, grid=(N // NB,),
        in_specs=[_nspec(8)] * 3 + [_nspec(), _wspec((HID, HID)),
                                    _wspec((HID, HID))],
        out_specs=[_nspec(8), _nspec(), _nspec()],
        out_shape=[jax.ShapeDtypeStruct((N, 8), jnp.float32)]
        + [jax.ShapeDtypeStruct((N, HID), jnp.float32)] * 2,
    )(x8, xp0, xp1, h1, w1a, w1b)


def _tc_out(h2, x8, xp0, xp1, wo1, bo1, wo2, bo2, wo3, bo3):
    def body(h2_r, x8_r, xp0_r, xp1_r, wo1_r, bo1_r, wo2_r, bo2_r,
             wo3_r, bo3_r, ho_r, xo_r):
        t = _silu(jnp.dot(h2_r[...], wo1_r[...],
                          preferred_element_type=jnp.float32, precision=lax.Precision.HIGHEST) + bo1_r[...])
        t = _silu(jnp.dot(t, wo2_r[...],
                          preferred_element_type=jnp.float32, precision=lax.Precision.HIGHEST) + bo2_r[...])
        ho_r[...] = jnp.dot(t, wo3_r[...],
                            preferred_element_type=jnp.float32, precision=lax.Precision.HIGHEST) + bo3_r[...]
        xo_r[...] = x8_r[...] + (xp0_r[...] + xp1_r[...]) * 0.01

    return pl.pallas_call(
        body, grid=(N // NB,),
        in_specs=[_nspec()] + [_nspec(8)] * 3
        + [_wspec((HID, HID)), _wspec((1, HID))] * 3,
        out_specs=[_nspec(), _nspec(8)],
        out_shape=[jax.ShapeDtypeStruct((N, HID), jnp.float32),
                   jax.ShapeDtypeStruct((N, 8), jnp.float32)],
    )(h2, x8, xp0, xp1, wo1, bo1, wo2, bo2, wo3, bo3)


def _row(b):
    return b.reshape(1, -1)


def kernel(h, x, edge_index, params):
    row = edge_index[0].astype(jnp.int32)
    col = edge_index[1].astype(jnp.int32)
    pad = E_PAD - E
    rowg = jnp.concatenate([row, jnp.zeros((pad,), jnp.int32)])
    colg = jnp.concatenate([col, jnp.zeros((pad,), jnp.int32)])
    rows_s = jnp.concatenate([row, jnp.full((pad,), N, jnp.int32)])
    x8 = jnp.pad(x, ((0, 0), (0, 5)))

    (we1, be1), (we2, be2) = params['embedding']
    (wo1, bo1), (wo2, bo2), (wo3, bo3) = params['embedding_out']

    def split_edge_w(mlp):
        (w1, b1), (w2, b2) = mlp
        return (w1[:HID], w1[HID:2 * HID], _row(w1[2 * HID]),
                _row(w1[2 * HID + 1]), _row(b1), w2, _row(b2))

    blk0, blk1 = params['blocks']
    e0 = split_edge_w(blk0['gcls'][0]['edge_mlp'])
    c0 = split_edge_w(blk0['coord_mlp'])
    (n0w1, n0b1), (n0w2, n0b2) = blk0['gcls'][0]['node_mlp']
    e1 = split_edge_w(blk1['gcls'][0]['edge_mlp'])
    c1 = split_edge_w(blk1['coord_mlp'])
    (n1w1, n1b1), (n1w2, n1b2) = blk1['gcls'][0]['node_mlp']
    wl0 = blk0['coord_last_W'].reshape(1, HID)
    wl1 = blk1['coord_last_W'].reshape(1, HID)

    h0, ha, hb = _tc_pre0(h, we1, _row(be1), we2, _row(be2), e0[0], e0[1])
    z, d8f = _gather_x(rowg, colg, ha, hb, x8.reshape(-1))
    d8 = d8f.reshape(E_PAD, 8)
    m, r8 = _tc_edge(True, z, d8, None, e0[2], e0[3], e0[4], e0[5], e0[6])
    p = _scatter_h(rows_s, m)[:, :N]
    h1, ca, cb = _tc_node(h0, p[0], p[1], n0w1[:HID], n0w1[HID:],
                          _row(n0b1), n0w2, _row(n0b2), c0[0], c0[1])
    zc = _gather(rowg, colg, ca, cb)
    t8 = _tc_coord(True, zc, d8, None, c0[2], c0[3], c0[4], c0[5], c0[6], wl0)
    xp = _scatter_x(rows_s, t8)[:, :N]

    x8_1, ha1, hb1 = _tc_pre1(x8, xp[0], xp[1], h1, e1[0], e1[1])
    z1, d8f_1 = _gather_x(rowg, colg, ha1, hb1, x8_1.reshape(-1))
    d8_1 = d8f_1.reshape(E_PAD, 8)
    m1 = _tc_edge(False, z1, d8_1, r8, e1[2], e1[3], e1[4], e1[5], e1[6])
    p1 = _scatter_h(rows_s, m1)[:, :N]
    h2, ca1, cb1 = _tc_node(h1, p1[0], p1[1], n1w1[:HID], n1w1[HID:],
                            _row(n1b1), n1w2, _row(n1b2), c1[0], c1[1])
    zc1 = _gather(rowg, colg, ca1, cb1)
    t81 = _tc_coord(False, zc1, d8_1, r8, c1[2], c1[3], c1[4], c1[5],
                    c1[6], wl1)
    xp1 = _scatter_x(rows_s, t81)[:, :N]

    hout, xo8 = _tc_out(h2, x8_1, xp1[0], xp1[1], wo1, _row(bo1),
                        wo2, _row(bo2), wo3, _row(bo3))
    return hout, xo8[:, :3]
